# baseline (speedup 1.0000x reference)
import functools

import jax
import jax.numpy as jnp
from jax import lax
from jax.experimental import pallas as pl
from jax.experimental.pallas import tpu as pltpu
from jax.experimental.pallas import tpu_sc as plsc

NC = 2
NS = 16
CH = 125
G = 128
F32 = jnp.float32


def _bn(h):
    m = jnp.mean(h, axis=0, keepdims=True)
    v = jnp.mean((h - m) ** 2, axis=0, keepdims=True)
    return (h - m) * lax.rsqrt(v + 1e-5) + 1e-4


def _mm(a, b):
    return jnp.dot(a, b, preferred_element_type=F32)


def _sc_mesh():
    return plsc.VectorSubcoreMesh(core_axis_name="c", subcore_axis_name="s")


def _pad_rows(n):
    return ((n + NS * 8 - 1) // (NS * 8)) * (NS * 8)


def _make_deg_kernel(n, e):
    nt = NC * NS
    nchk = (e // nt) // CH
    np_ = _pad_rows(n)
    nfull, rem = divmod(CH, 16)
    mesh = _sc_mesh()

    @functools.partial(
        pl.kernel,
        out_type=jax.ShapeDtypeStruct((nt, np_), F32),
        mesh=mesh,
        compiler_params=pltpu.CompilerParams(needs_layout_passes=False),
        scratch_types=[
            pltpu.VMEM((nchk, CH), jnp.int32),
            pltpu.VMEM((np_,), F32),
        ],
    )
    def deg_kernel(ei_hbm, out_hbm, idx, cnt):
        c = lax.axis_index("c")
        s = lax.axis_index("s")
        tid = c * NS + s

        def zero(i, _):
            cnt[pl.ds(i * 16, 16)] = jnp.zeros((16,), F32)
            return 0
        lax.fori_loop(0, np_ // 16, zero, 0)
        pltpu.sync_copy(ei_hbm.at[pl.ds(tid * nchk, nchk)], idx)
        ones = jnp.ones((16,), F32)
        tailmask = lax.broadcasted_iota(jnp.int32, (16,), 0) >= (16 - rem)

        def body(i, _):
            for jj in range(nfull):
                plsc.addupdate_scatter(cnt, [idx[i, pl.ds(jj * 16, 16)]],
                                       ones)
            if rem:
                plsc.addupdate_scatter(cnt, [idx[i, pl.ds(CH - 16, 16)]],
                                       ones, mask=tailmask)
            return 0
        lax.fori_loop(0, nchk, body, 0)
        pltpu.sync_copy(cnt, out_hbm.at[tid])

    return deg_kernel


def _make_prop_kernel(n, h, e):
    nt = NC * NS
    hh = h // NC
    nchk = (e // NS) // CH
    np_ = _pad_rows(n)
    zr = np_ // NS
    mesh = _sc_mesh()

    @functools.partial(
        pl.kernel,
        out_type=jax.ShapeDtypeStruct((NC * np_, hh), F32),
        mesh=mesh,
        compiler_params=pltpu.CompilerParams(use_tc_tiling_on_sc=False),
        scratch_types=[
            pltpu.VMEM((nchk, CH), jnp.int32),
            pltpu.VMEM((nchk, CH), jnp.int32),
            [pltpu.VMEM((CH, hh), F32)] * 4,
            pltpu.VMEM((128, hh), F32),
            pltpu.VMEM_SHARED((np_, hh), F32),
            [pltpu.SemaphoreType.DMA] * 4,
            [pltpu.SemaphoreType.DMA] * 4,
        ],
    )
    def prop_kernel(hp_hbm, rowa_hbm, rowb_hbm, col_hbm, out_hbm,
                    idxr, idxc, gbufs, zb, acc, gsems, ssems):
        c = lax.axis_index("c")
        s = lax.axis_index("s")
        start = s * zr

        def gath(j, bi):
            pltpu.async_copy(hp_hbm.at[idxr.at[j]], gbufs[bi], gsems[bi])

        def gath_wait(j, bi):
            pltpu.make_async_copy(
                hp_hbm.at[idxr.at[j]], gbufs[bi], gsems[bi]).wait()

        def scat(j, bi):
            pltpu.async_copy(gbufs[bi], acc.at[idxc.at[j]], ssems[bi],
                             add=True)

        def scat_wait(j, bi):
            pltpu.make_async_copy(
                gbufs[bi], acc.at[idxc.at[j]], ssems[bi]).wait()

        @pl.when(c == 0)
        def _():
            pltpu.sync_copy(rowa_hbm.at[pl.ds(s * nchk, nchk)], idxr)

        @pl.when(c == 1)
        def _():
            pltpu.sync_copy(rowb_hbm.at[pl.ds(s * nchk, nchk)], idxr)
        pltpu.sync_copy(col_hbm.at[pl.ds(s * nchk, nchk)], idxc)

        gath(0, 0)
        gath(1, 1)

        def fill_z(i, _):
            for jj in range(hh // 16):
                zb[i, pl.ds(jj * 16, 16)] = jnp.zeros((16,), F32)
            return 0
        lax.fori_loop(0, 128, fill_z, 0)
        nfull, rem = divmod(zr, 128)
        for k in range(nfull):
            pltpu.sync_copy(zb, acc.at[pl.ds(start + k * 128, 128)])
        if rem:
            pltpu.sync_copy(zb.at[pl.ds(0, rem)],
                            acc.at[pl.ds(start + nfull * 128, rem)])
        plsc.subcore_barrier()

        def body(k, _):
            j0 = 4 * k
            for i in range(4):
                j = j0 + i
                bi = i
                pbi = (i + 2) % 4

                @pl.when(j >= 2)
                def _(j=j, pbi=pbi):
                    scat_wait(j - 2, pbi)

                @pl.when(j + 2 < nchk)
                def _(j=j, pbi=pbi):
                    gath(j + 2, pbi)
                gath_wait(j, bi)
                scat(j, bi)
            return 0
        lax.fori_loop(0, nchk // 4, body, 0)
        for j in range(nchk - 2, nchk):
            scat_wait(j, j % 4)
        plsc.subcore_barrier()
        pltpu.sync_copy(acc.at[pl.ds(start, zr)],
                        out_hbm.at[pl.ds(c * np_ + start, zr)])

    return prop_kernel


def _prep1(x, w_feat, cnt, w1, n, h):
    def body(x_ref, wf_ref, cnt_ref, w1_ref, o_ref, dis_ref):
        xb = _bn(x_ref[...])
        h0 = jnp.maximum(_mm(xb, wf_ref[...]), 0.0)
        degr = jnp.sum(cnt_ref[...], axis=0, keepdims=True) + 1.0
        dis = lax.transpose(lax.rsqrt(degr), (1, 0))[0:n]
        dis_ref[...] = dis
        o_ref[...] = dis * _mm(_bn(h0), w1_ref[...])
    return pl.pallas_call(
        body,
        out_shape=[jax.ShapeDtypeStruct((n, h), F32),
                   jax.ShapeDtypeStruct((n, 1), F32)],
    )(x, w_feat, cnt, w1)


def _unpack_s(s_ref, n, np_):
    return jnp.concatenate([s_ref[0:n], s_ref[np_:np_ + n]], axis=1)


def _tail_bn_mm_scale(s, hp, b, dis, w, n, h):
    np_ = _pad_rows(n)

    def body(s_ref, hp_ref, b_ref, dis_ref, w_ref, o_ref):
        t = _unpack_s(s_ref, n, np_) + hp_ref[...]
        h1 = jnp.maximum(dis_ref[...] * t + b_ref[...], 0.0)
        o_ref[...] = dis_ref[...] * _mm(_bn(h1), w_ref[...])
    return pl.pallas_call(
        body, out_shape=jax.ShapeDtypeStruct((n, h), F32),
    )(s, hp, b, dis, w)


def _head(s, hp2, b3, dis, batch2, w_fc, b_fc, w_cls, b_cls, n, h, c_out):
    np_ = _pad_rows(n)

    def body(s_ref, hp_ref, b3_ref, dis_ref, batch_ref,
             wfc_ref, bfc_ref, wcls_ref, bcls_ref, o_ref):
        t = _unpack_s(s_ref, n, np_) + hp_ref[...]
        h3 = jnp.maximum(dis_ref[...] * t + b3_ref[...], 0.0)
        seg = lax.broadcasted_iota(jnp.int32, (G, n), 0)
        onehot = (seg == batch_ref[...]).astype(F32)
        g = _mm(onehot, h3)
        g = _bn(g)
        g = jnp.maximum(_mm(g, wfc_ref[...]) + bfc_ref[...], 0.0)
        g = _bn(g)
        logits = _mm(g, wcls_ref[...]) + bcls_ref[...]
        mx = jnp.max(logits, axis=-1, keepdims=True)
        lse = jnp.log(jnp.sum(jnp.exp(logits - mx), axis=-1, keepdims=True)) + mx
        o_ref[...] = logits - lse
    return pl.pallas_call(
        body, out_shape=jax.ShapeDtypeStruct((G, c_out), F32),
    )(s, hp2, b3, dis, batch2, w_fc, b_fc, w_cls, b_cls)


def kernel(x, edge_index, batch, W_feat, W1, b1, W2, b2, W3, b3,
           W_fc, b_fc, W_cls, b_cls):
    n, _ = x.shape
    h = W_feat.shape[1]
    e = edge_index.shape[1]
    c_out = W_cls.shape[1]

    row = edge_index[0]
    rowa = (2 * row).reshape(e // CH, CH)
    rowb = rowa + 1
    col2 = edge_index[1].reshape(e // CH, CH)
    batch2 = batch.reshape(1, n)

    cnt = _make_deg_kernel(n, e)(edge_index.reshape(2 * (e // CH), CH))
    hp, dis = _prep1(x, W_feat, cnt, W1, n, h)

    prop = _make_prop_kernel(n, h, e)
    s = prop(hp.reshape(2 * n, h // 2), rowa, rowb, col2)
    hp = _tail_bn_mm_scale(s, hp, b1.reshape(1, h), dis, W2, n, h)
    s = prop(hp.reshape(2 * n, h // 2), rowa, rowb, col2)
    hp = _tail_bn_mm_scale(s, hp, b2.reshape(1, h), dis, W3, n, h)
    s = prop(hp.reshape(2 * n, h // 2), rowa, rowb, col2)
    return _head(s, hp, b3.reshape(1, h), dis, batch2,
                 W_fc, b_fc.reshape(1, h), W_cls, b_cls.reshape(1, c_out),
                 n, h, c_out)


# speedup vs baseline: 1.0588x; 1.0588x over previous
import functools

import jax
import jax.numpy as jnp
from jax import lax
from jax.experimental import pallas as pl
from jax.experimental.pallas import tpu as pltpu
from jax.experimental.pallas import tpu_sc as plsc

NC = 2
NS = 16
CH = 125
G = 128
F32 = jnp.float32


def _bn(h):
    m = jnp.mean(h, axis=0, keepdims=True)
    v = jnp.mean((h - m) ** 2, axis=0, keepdims=True)
    return (h - m) * lax.rsqrt(v + 1e-5) + 1e-4


def _mm(a, b):
    return jnp.dot(a, b, preferred_element_type=F32)


def _sc_mesh():
    return plsc.VectorSubcoreMesh(core_axis_name="c", subcore_axis_name="s")


def _pad_rows(n):
    return ((n + NS * 8 - 1) // (NS * 8)) * (NS * 8)


def _make_deg_kernel(n, e):
    nt = NC * NS
    ept = e // nt
    np_ = _pad_rows(n)
    mesh = _sc_mesh()

    @functools.partial(
        pl.kernel,
        out_type=jax.ShapeDtypeStruct((nt, np_), F32),
        mesh=mesh,
        compiler_params=pltpu.CompilerParams(needs_layout_passes=False),
        scratch_types=[
            pltpu.VMEM((ept,), jnp.int32),
            pltpu.VMEM((np_,), F32),
        ],
    )
    def deg_kernel(row_hbm, out_hbm, idx, cnt):
        c = lax.axis_index("c")
        s = lax.axis_index("s")
        tid = c * NS + s

        def zero(i, _):
            cnt[pl.ds(i * 16, 16)] = jnp.zeros((16,), F32)
            return 0
        lax.fori_loop(0, np_ // 16, zero, 0)
        pltpu.sync_copy(row_hbm.at[pl.ds(tid * ept, ept)], idx)
        ones = jnp.ones((16,), F32)

        def body(i, _):
            plsc.addupdate_scatter(cnt, [idx[pl.ds(i * 16, 16)]], ones)
            return 0
        lax.fori_loop(0, ept // 16, body, 0)
        pltpu.sync_copy(cnt, out_hbm.at[tid])

    return deg_kernel


def _make_prop_kernel(n, h, e):
    nt = NC * NS
    hh = h // NC
    nchk = (e // NS) // CH
    np_ = _pad_rows(n)
    zr = np_ // NS
    mesh = _sc_mesh()

    @functools.partial(
        pl.kernel,
        out_type=jax.ShapeDtypeStruct((NC * np_, hh), F32),
        mesh=mesh,
        compiler_params=pltpu.CompilerParams(use_tc_tiling_on_sc=False),
        scratch_types=[
            pltpu.VMEM((nchk, CH), jnp.int32),
            pltpu.VMEM((nchk, CH), jnp.int32),
            [pltpu.VMEM((CH, hh), F32)] * 4,
            pltpu.VMEM((128, hh), F32),
            pltpu.VMEM_SHARED((np_, hh), F32),
            [pltpu.SemaphoreType.DMA] * 4,
            [pltpu.SemaphoreType.DMA] * 4,
        ],
    )
    def prop_kernel(hp_hbm, rowa_hbm, rowb_hbm, col_hbm, out_hbm,
                    idxr, idxc, gbufs, zb, acc, gsems, ssems):
        c = lax.axis_index("c")
        s = lax.axis_index("s")
        start = s * zr

        def gath(j, bi):
            pltpu.async_copy(hp_hbm.at[idxr.at[j]], gbufs[bi], gsems[bi])

        def gath_wait(j, bi):
            pltpu.make_async_copy(
                hp_hbm.at[idxr.at[j]], gbufs[bi], gsems[bi]).wait()

        def scat(j, bi):
            pltpu.async_copy(gbufs[bi], acc.at[idxc.at[j]], ssems[bi],
                             add=True)

        def scat_wait(j, bi):
            pltpu.make_async_copy(
                gbufs[bi], acc.at[idxc.at[j]], ssems[bi]).wait()

        @pl.when(c == 0)
        def _():
            pltpu.sync_copy(rowa_hbm.at[pl.ds(s * nchk, nchk)], idxr)

        @pl.when(c == 1)
        def _():
            pltpu.sync_copy(rowb_hbm.at[pl.ds(s * nchk, nchk)], idxr)
        pltpu.sync_copy(col_hbm.at[pl.ds(s * nchk, nchk)], idxc)

        gath(0, 0)
        gath(1, 1)

        def fill_z(i, _):
            for jj in range(hh // 16):
                zb[i, pl.ds(jj * 16, 16)] = jnp.zeros((16,), F32)
            return 0
        lax.fori_loop(0, 128, fill_z, 0)
        nfull, rem = divmod(zr, 128)
        for k in range(nfull):
            pltpu.sync_copy(zb, acc.at[pl.ds(start + k * 128, 128)])
        if rem:
            pltpu.sync_copy(zb.at[pl.ds(0, rem)],
                            acc.at[pl.ds(start + nfull * 128, rem)])
        plsc.subcore_barrier()

        def body(k, _):
            j0 = 4 * k
            for i in range(4):
                j = j0 + i
                bi = i
                pbi = (i + 2) % 4

                @pl.when(j >= 2)
                def _(j=j, pbi=pbi):
                    scat_wait(j - 2, pbi)

                @pl.when(j + 2 < nchk)
                def _(j=j, pbi=pbi):
                    gath(j + 2, pbi)
                gath_wait(j, bi)
                scat(j, bi)
            return 0
        lax.fori_loop(0, nchk // 4, body, 0)
        for j in range(nchk - 2, nchk):
            scat_wait(j, j % 4)
        plsc.subcore_barrier()
        pltpu.sync_copy(acc.at[pl.ds(start, zr)],
                        out_hbm.at[pl.ds(c * np_ + start, zr)])

    return prop_kernel


def _prep1(x, w_feat, cnt, w1, n, h):
    def body(x_ref, wf_ref, cnt_ref, w1_ref, o_ref, dis_ref):
        xb = _bn(x_ref[...])
        h0 = jnp.maximum(_mm(xb, wf_ref[...]), 0.0)
        degr = jnp.sum(cnt_ref[...], axis=0, keepdims=True) + 1.0
        dis = lax.transpose(lax.rsqrt(degr), (1, 0))[0:n]
        dis_ref[...] = dis
        o_ref[...] = dis * _mm(_bn(h0), w1_ref[...])
    return pl.pallas_call(
        body,
        out_shape=[jax.ShapeDtypeStruct((n, h), F32),
                   jax.ShapeDtypeStruct((n, 1), F32)],
    )(x, w_feat, cnt, w1)


def _unpack_s(s_ref, n, np_):
    hh = s_ref.shape[1] // 2
    spa = s_ref[0:np_ // 2]
    spb = s_ref[np_ // 2:np_]
    t_even = jnp.concatenate([spa[:, 0:hh], spb[:, 0:hh]], axis=1)
    t_odd = jnp.concatenate([spa[:, hh:], spb[:, hh:]], axis=1)
    t = jnp.stack([t_even, t_odd], axis=1).reshape(np_, 2 * hh)
    return t[0:n]


def _tail_bn_mm_scale(s, hp, b, dis, w, n, h):
    np_ = _pad_rows(n)

    def body(s_ref, hp_ref, b_ref, dis_ref, w_ref, o_ref):
        t = _unpack_s(s_ref, n, np_) + hp_ref[...]
        h1 = jnp.maximum(dis_ref[...] * t + b_ref[...], 0.0)
        o_ref[...] = dis_ref[...] * _mm(_bn(h1), w_ref[...])
    return pl.pallas_call(
        body, out_shape=jax.ShapeDtypeStruct((n, h), F32),
    )(s, hp, b, dis, w)


def _head(s, hp2, b3, dis, batch2, w_fc, b_fc, w_cls, b_cls, n, h, c_out):
    np_ = _pad_rows(n)

    def body(s_ref, hp_ref, b3_ref, dis_ref, batch_ref,
             wfc_ref, bfc_ref, wcls_ref, bcls_ref, o_ref):
        t = _unpack_s(s_ref, n, np_) + hp_ref[...]
        h3 = jnp.maximum(dis_ref[...] * t + b3_ref[...], 0.0)
        seg = lax.broadcasted_iota(jnp.int32, (G, n), 0)
        onehot = (seg == batch_ref[...]).astype(F32)
        g = _mm(onehot, h3)
        g = _bn(g)
        g = jnp.maximum(_mm(g, wfc_ref[...]) + bfc_ref[...], 0.0)
        g = _bn(g)
        logits = _mm(g, wcls_ref[...]) + bcls_ref[...]
        mx = jnp.max(logits, axis=-1, keepdims=True)
        lse = jnp.log(jnp.sum(jnp.exp(logits - mx), axis=-1, keepdims=True)) + mx
        o_ref[...] = logits - lse
    return pl.pallas_call(
        body, out_shape=jax.ShapeDtypeStruct((G, c_out), F32),
    )(s, hp2, b3, dis, batch2, w_fc, b_fc, w_cls, b_cls)


def kernel(x, edge_index, batch, W_feat, W1, b1, W2, b2, W3, b3,
           W_fc, b_fc, W_cls, b_cls):
    n, _ = x.shape
    h = W_feat.shape[1]
    e = edge_index.shape[1]
    c_out = W_cls.shape[1]

    row = edge_index[0]
    rowa = (2 * row).reshape(e // CH, CH)
    rowb = rowa + 1
    col2 = edge_index[1].reshape(e // CH, CH)
    batch2 = batch.reshape(1, n)

    cnt = _make_deg_kernel(n, e)(row)
    hp, dis = _prep1(x, W_feat, cnt, W1, n, h)

    np_ = _pad_rows(n)
    prop = _make_prop_kernel(n, h, e)
    s = prop(hp.reshape(2 * n, h // 2), rowa, rowb, col2).reshape(np_, h)
    hp = _tail_bn_mm_scale(s, hp, b1.reshape(1, h), dis, W2, n, h)
    s = prop(hp.reshape(2 * n, h // 2), rowa, rowb, col2).reshape(np_, h)
    hp = _tail_bn_mm_scale(s, hp, b2.reshape(1, h), dis, W3, n, h)
    s = prop(hp.reshape(2 * n, h // 2), rowa, rowb, col2).reshape(np_, h)
    return _head(s, hp, b3.reshape(1, h), dis, batch2,
                 W_fc, b_fc.reshape(1, h), W_cls, b_cls.reshape(1, c_out),
                 n, h, c_out)


# speedup vs baseline: 1.0598x; 1.0009x over previous
import functools

import jax
import jax.numpy as jnp
from jax import lax
from jax.experimental import pallas as pl
from jax.experimental.pallas import tpu as pltpu
from jax.experimental.pallas import tpu_sc as plsc

NC = 2
NS = 16
CH = 125
G = 128
F32 = jnp.float32


def _bn(h):
    m = jnp.mean(h, axis=0, keepdims=True)
    v = jnp.mean((h - m) ** 2, axis=0, keepdims=True)
    return (h - m) * lax.rsqrt(v + 1e-5) + 1e-4


def _mm(a, b):
    return jnp.dot(a, b, preferred_element_type=F32)


def _sc_mesh():
    return plsc.VectorSubcoreMesh(core_axis_name="c", subcore_axis_name="s")


def _pad_rows(n):
    return ((n + NS * 8 - 1) // (NS * 8)) * (NS * 8)


def _make_deg_kernel(n, e):
    nt = NC * NS
    ept = e // nt
    np_ = _pad_rows(n)
    mesh = _sc_mesh()

    @functools.partial(
        pl.kernel,
        out_type=jax.ShapeDtypeStruct((nt, np_), F32),
        mesh=mesh,
        compiler_params=pltpu.CompilerParams(needs_layout_passes=False),
        scratch_types=[
            pltpu.VMEM((ept,), jnp.int32),
            pltpu.VMEM((np_,), F32),
        ],
    )
    def deg_kernel(row_hbm, out_hbm, idx, cnt):
        c = lax.axis_index("c")
        s = lax.axis_index("s")
        tid = c * NS + s

        def zero(i, _):
            cnt[pl.ds(i * 16, 16)] = jnp.zeros((16,), F32)
            return 0
        lax.fori_loop(0, np_ // 16, zero, 0)
        pltpu.sync_copy(row_hbm.at[pl.ds(tid * ept, ept)], idx)
        ones = jnp.ones((16,), F32)

        def body(i, _):
            plsc.addupdate_scatter(cnt, [idx[pl.ds(i * 16, 16)]], ones)
            return 0
        lax.fori_loop(0, ept // 16, body, 0)
        pltpu.sync_copy(cnt, out_hbm.at[tid])

    return deg_kernel


def _make_prop_kernel(n, h, e):
    hh = h // NC
    nchk = (e // NS) // CH
    np_ = _pad_rows(n)
    zr = np_ // NS
    mesh = _sc_mesh()

    @functools.partial(
        pl.kernel,
        out_type=jax.ShapeDtypeStruct((NC * np_, hh), F32),
        mesh=mesh,
        compiler_params=pltpu.CompilerParams(use_tc_tiling_on_sc=False),
        scratch_types=[
            pltpu.VMEM((nchk, CH), jnp.int32),
            pltpu.VMEM((nchk, CH), jnp.int32),
            [pltpu.VMEM((CH, hh), F32)] * 4,
            pltpu.VMEM((128, hh), F32),
            pltpu.VMEM_SHARED((np_, hh), F32),
            [pltpu.SemaphoreType.DMA] * 4,
            [pltpu.SemaphoreType.DMA] * 4,
        ],
    )
    def prop_kernel(hp_hbm, rowa_hbm, rowb_hbm, col_hbm, out_hbm,
                    idxr, idxc, gbufs, zb, acc, gsems, ssems):
        c = lax.axis_index("c")
        s = lax.axis_index("s")
        start = s * zr

        def gath(j, bi):
            pltpu.async_copy(hp_hbm.at[idxr.at[j]], gbufs[bi], gsems[bi])

        def gath_wait(j, bi):
            pltpu.make_async_copy(
                hp_hbm.at[idxr.at[j]], gbufs[bi], gsems[bi]).wait()

        def scat(j, bi):
            pltpu.async_copy(gbufs[bi], acc.at[idxc.at[j]], ssems[bi],
                             add=True)

        def scat_wait(j, bi):
            pltpu.make_async_copy(
                gbufs[bi], acc.at[idxc.at[j]], ssems[bi]).wait()

        @pl.when(c == 0)
        def _():
            pltpu.sync_copy(rowa_hbm.at[pl.ds(s * nchk, nchk)], idxr)

        @pl.when(c == 1)
        def _():
            pltpu.sync_copy(rowb_hbm.at[pl.ds(s * nchk, nchk)], idxr)
        pltpu.sync_copy(col_hbm.at[pl.ds(s * nchk, nchk)], idxc)

        gath(0, 0)
        gath(1, 1)

        def fill_z(i, _):
            for jj in range(hh // 16):
                zb[i, pl.ds(jj * 16, 16)] = jnp.zeros((16,), F32)
            return 0
        lax.fori_loop(0, 128, fill_z, 0)
        nfull, rem = divmod(zr, 128)
        for k in range(nfull):
            pltpu.sync_copy(zb, acc.at[pl.ds(start + k * 128, 128)])
        if rem:
            pltpu.sync_copy(zb.at[pl.ds(0, rem)],
                            acc.at[pl.ds(start + nfull * 128, rem)])
        plsc.subcore_barrier()

        def body(k, _):
            j0 = 4 * k
            for i in range(4):
                j = j0 + i
                bi = i
                pbi = (i + 2) % 4

                @pl.when(j >= 2)
                def _(j=j, pbi=pbi):
                    scat_wait(j - 2, pbi)

                @pl.when(j + 2 < nchk)
                def _(j=j, pbi=pbi):
                    gath(j + 2, pbi)
                gath_wait(j, bi)
                scat(j, bi)
            return 0
        lax.fori_loop(0, nchk // 4, body, 0)
        for j in range(nchk - 2, nchk):
            scat_wait(j, j % 4)
        plsc.subcore_barrier()
        pltpu.sync_copy(acc.at[pl.ds(start, zr)],
                        out_hbm.at[pl.ds(c * np_ + start, zr)])

    return prop_kernel


def _prep1(x, w_feat, cnt, w1, n, h):
    def body(x_ref, wf_ref, cnt_ref, w1_ref, o_ref, dis_ref):
        xb = _bn(x_ref[...])
        h0 = jnp.maximum(_mm(xb, wf_ref[...]), 0.0)
        degr = jnp.sum(cnt_ref[...], axis=0, keepdims=True) + 1.0
        dis = lax.transpose(lax.rsqrt(degr), (1, 0))[0:n]
        dis_ref[...] = dis
        o_ref[...] = dis * _mm(_bn(h0), w1_ref[...])
    return pl.pallas_call(
        body,
        out_shape=[jax.ShapeDtypeStruct((n, h), F32),
                   jax.ShapeDtypeStruct((n, 1), F32)],
    )(x, w_feat, cnt, w1)


def _unpack_s(s_ref, n, np_):
    hh = s_ref.shape[1] // 2
    spa = s_ref[0:np_ // 2]
    spb = s_ref[np_ // 2:np_]
    t_even = jnp.concatenate([spa[:, 0:hh], spb[:, 0:hh]], axis=1)
    t_odd = jnp.concatenate([spa[:, hh:], spb[:, hh:]], axis=1)
    t = jnp.stack([t_even, t_odd], axis=1).reshape(np_, 2 * hh)
    return t[0:n]


def _tail_bn_mm_scale(s, hp, b, dis, w, n, h):
    np_ = _pad_rows(n)

    def body(s_ref, hp_ref, b_ref, dis_ref, w_ref, o_ref):
        t = _unpack_s(s_ref, n, np_) + hp_ref[...]
        h1 = jnp.maximum(dis_ref[...] * t + b_ref[...], 0.0)
        o_ref[...] = dis_ref[...] * _mm(_bn(h1), w_ref[...])
    return pl.pallas_call(
        body, out_shape=jax.ShapeDtypeStruct((n, h), F32),
    )(s, hp, b, dis, w)


def _head(s, hp2, b3, dis, batch2, w_fc, b_fc, w_cls, b_cls, n, h, c_out):
    np_ = _pad_rows(n)

    def body(s_ref, hp_ref, b3_ref, dis_ref, batch_ref,
             wfc_ref, bfc_ref, wcls_ref, bcls_ref, o_ref):
        t = _unpack_s(s_ref, n, np_) + hp_ref[...]
        h3 = jnp.maximum(dis_ref[...] * t + b3_ref[...], 0.0)
        seg = lax.broadcasted_iota(jnp.int32, (G, n), 0)
        onehot = (seg == batch_ref[...]).astype(F32)
        g = _mm(onehot, h3)
        g = _bn(g)
        g = jnp.maximum(_mm(g, wfc_ref[...]) + bfc_ref[...], 0.0)
        g = _bn(g)
        logits = _mm(g, wcls_ref[...]) + bcls_ref[...]
        mx = jnp.max(logits, axis=-1, keepdims=True)
        lse = jnp.log(jnp.sum(jnp.exp(logits - mx), axis=-1, keepdims=True)) + mx
        o_ref[...] = logits - lse
    return pl.pallas_call(
        body, out_shape=jax.ShapeDtypeStruct((G, c_out), F32),
    )(s, hp2, b3, dis, batch2, w_fc, b_fc, w_cls, b_cls)


def kernel(x, edge_index, batch, W_feat, W1, b1, W2, b2, W3, b3,
           W_fc, b_fc, W_cls, b_cls):
    n, _ = x.shape
    h = W_feat.shape[1]
    e = edge_index.shape[1]
    c_out = W_cls.shape[1]

    row = edge_index[0]
    rowa = (2 * row).reshape(e // CH, CH)
    rowb = rowa + 1
    col2 = edge_index[1].reshape(e // CH, CH)
    batch2 = batch.reshape(1, n)

    cnt = _make_deg_kernel(n, e)(row)
    hp, dis = _prep1(x, W_feat, cnt, W1, n, h)

    np_ = _pad_rows(n)
    prop = _make_prop_kernel(n, h, e)
    s = prop(hp.reshape(2 * n, h // 2), rowa, rowb, col2).reshape(np_, h)
    hp = _tail_bn_mm_scale(s, hp, b1.reshape(1, h), dis, W2, n, h)
    s = prop(hp.reshape(2 * n, h // 2), rowa, rowb, col2).reshape(np_, h)
    hp = _tail_bn_mm_scale(s, hp, b2.reshape(1, h), dis, W3, n, h)
    s = prop(hp.reshape(2 * n, h // 2), rowa, rowb, col2).reshape(np_, h)
    return _head(s, hp, b3.reshape(1, h), dis, batch2,
                 W_fc, b_fc.reshape(1, h), W_cls, b_cls.reshape(1, c_out),
                 n, h, c_out)
